# Initial kernel scaffold; baseline (speedup 1.0000x reference)
import functools

import jax
import jax.numpy as jnp
from jax import lax
from jax.experimental import pallas as pl
from jax.experimental.pallas import tpu as pltpu
from jax.experimental.pallas import tpu_sc as plsc

VOCAB = 100000
D = 128
N = 1024 * 200
NC, NS, LANES = 2, 16, 16
NW = NC * NS
PER_W = N // NW
CHUNK = 128
NCHUNK = PER_W // CHUNK


def _emb_body(ids_hbm, mask_hbm, table_hbm, out_hbm, idx_v, mask_v, buf, sem):
    c = lax.axis_index("c")
    s = lax.axis_index("s")
    wid = s * NC + c
    base = wid * PER_W

    pltpu.sync_copy(ids_hbm.at[pl.ds(base, PER_W)], idx_v)
    pltpu.sync_copy(mask_hbm.at[pl.ds(base, PER_W)], mask_v)

    def chunk_body(g, carry):
        off = g * CHUNK
        pltpu.async_copy(table_hbm.at[idx_v.at[pl.ds(off, CHUNK)]], buf, sem).wait()

        flags = []
        for v in range(CHUNK // LANES):
            ids = idx_v[pl.ds(off + v * LANES, LANES)]
            mk = mask_v[pl.ds(off + v * LANES, LANES)]
            flags.append(jnp.any((ids == 0) | (mk != 1.0)))
        need = functools.reduce(jnp.logical_or, flags)

        @pl.when(need)
        def _fixup():
            def fix_row(r, cr):
                pos = jnp.full((LANES,), off + r, dtype=jnp.int32)
                idb = plsc.load_gather(idx_v, [pos])
                mkb = plsc.load_gather(mask_v, [pos])
                scale = jnp.where(idb == 0, jnp.zeros((LANES,), jnp.float32), mkb)
                for j in range(D // LANES):
                    buf[r, pl.ds(j * LANES, LANES)] = (
                        buf[r, pl.ds(j * LANES, LANES)] * scale
                    )
                return cr
            lax.fori_loop(0, CHUNK, fix_row, 0)

        pltpu.sync_copy(buf, out_hbm.at[pl.ds(base + off, CHUNK)])
        return carry

    lax.fori_loop(0, NCHUNK, chunk_body, 0)


@jax.jit
def _emb_call(ids_flat, mask_flat, table):
    mesh = plsc.VectorSubcoreMesh(core_axis_name="c", subcore_axis_name="s")
    kern = pl.kernel(
        _emb_body,
        out_type=jax.ShapeDtypeStruct((N, D), jnp.float32),
        mesh=mesh,
        scratch_types=[
            pltpu.VMEM((PER_W,), jnp.int32),
            pltpu.VMEM((PER_W,), jnp.float32),
            pltpu.VMEM((CHUNK, D), jnp.float32),
            pltpu.SemaphoreType.DMA,
        ],
    )
    return kern(ids_flat, mask_flat, table)


def kernel(input_ids, attention_mask, table):
    B, L = input_ids.shape
    out = _emb_call(
        input_ids.reshape(-1).astype(jnp.int32),
        attention_mask.reshape(-1).astype(jnp.float32),
        table,
    )
    return out.reshape(B, L, D)


# speedup vs baseline: 5.6958x; 5.6958x over previous
import functools

import jax
import jax.numpy as jnp
from jax import lax
from jax.experimental import pallas as pl
from jax.experimental.pallas import tpu as pltpu
from jax.experimental.pallas import tpu_sc as plsc

VOCAB = 100000
D = 128
N = 1024 * 200
NC, NS, LANES = 2, 16, 16
NW = NC * NS
PER_W = N // NW
CHUNK = 128
NCHUNK = PER_W // CHUNK


def _emb_body(ids_hbm, mask_hbm, table_hbm, out_hbm, idx_v, mask_v, buf, sem):
    c = lax.axis_index("c")
    s = lax.axis_index("s")
    wid = s * NC + c
    base = wid * PER_W

    pltpu.sync_copy(ids_hbm.at[pl.ds(base, PER_W)], idx_v)
    pltpu.sync_copy(mask_hbm.at[pl.ds(base, PER_W)], mask_v)

    def chunk_body(g, carry):
        off = g * CHUNK
        pltpu.async_copy(table_hbm.at[idx_v.at[pl.ds(off, CHUNK)]], buf, sem).wait()

        flags = []
        for v in range(CHUNK // LANES):
            ids = idx_v[pl.ds(off + v * LANES, LANES)]
            mk = mask_v[pl.ds(off + v * LANES, LANES)]
            flags.append(jnp.any((ids == 0) | (mk != 1.0)))
        need = functools.reduce(jnp.logical_or, flags)

        @pl.when(need)
        def _fixup():
            def fix_row(r, cr):
                pos = jnp.full((LANES,), off + r, dtype=jnp.int32)
                idb = plsc.load_gather(idx_v, [pos])
                mkb = plsc.load_gather(mask_v, [pos])
                scale = jnp.where(idb == 0, jnp.zeros((LANES,), jnp.float32), mkb)
                for j in range(D // LANES):
                    buf[r, pl.ds(j * LANES, LANES)] = (
                        buf[r, pl.ds(j * LANES, LANES)] * scale
                    )
                return cr
            lax.fori_loop(0, CHUNK, fix_row, 0)

        pltpu.sync_copy(buf, out_hbm.at[pl.ds(base + off, CHUNK)])
        return carry

    lax.fori_loop(0, NCHUNK, chunk_body, 0)


@jax.jit
def _emb_call(ids_flat, mask_flat, table):
    mesh = plsc.VectorSubcoreMesh(core_axis_name="c", subcore_axis_name="s")
    kern = pl.kernel(
        _emb_body,
        out_type=jax.ShapeDtypeStruct((N, D), jnp.float32),
        mesh=mesh,
        compiler_params=pltpu.CompilerParams(needs_layout_passes=False),
        scratch_types=[
            pltpu.VMEM((PER_W,), jnp.int32),
            pltpu.VMEM((PER_W,), jnp.float32),
            pltpu.VMEM((CHUNK, D), jnp.float32),
            pltpu.SemaphoreType.DMA,
        ],
    )
    return kern(ids_flat, mask_flat, table)


def kernel(input_ids, attention_mask, table):
    B, L = input_ids.shape
    out = _emb_call(
        input_ids.reshape(-1).astype(jnp.int32),
        attention_mask.reshape(-1).astype(jnp.float32),
        table,
    )
    return out.reshape(B, L, D)


# speedup vs baseline: 7.9884x; 1.4025x over previous
import functools

import jax
import jax.numpy as jnp
from jax import lax
from jax.experimental import pallas as pl
from jax.experimental.pallas import tpu as pltpu
from jax.experimental.pallas import tpu_sc as plsc

VOCAB = 100000
D = 128
N = 1024 * 200
NC, NS, LANES = 2, 16, 16
NW = NC * NS
PER_W = N // NW
CHUNK = 128
NCHUNK = PER_W // CHUNK
NBUF = 5


def _emb_body(ids_hbm, mask_hbm, table_hbm, out_hbm, idx_v, mask_v, *rest):
    bufs = rest[:NBUF]
    gsems = rest[NBUF:2 * NBUF]
    wsems = rest[2 * NBUF:3 * NBUF]

    c = lax.axis_index("c")
    s = lax.axis_index("s")
    wid = s * NC + c
    base = wid * PER_W

    pltpu.sync_copy(ids_hbm.at[pl.ds(base, PER_W)], idx_v)
    pltpu.sync_copy(mask_hbm.at[pl.ds(base, PER_W)], mask_v)

    def gather_desc(g, b):
        off = g * CHUNK
        return pltpu.make_async_copy(
            table_hbm.at[idx_v.at[pl.ds(off, CHUNK)]], bufs[b], gsems[b]
        )

    def write_desc(g, b):
        off = g * CHUNK
        return pltpu.make_async_copy(
            bufs[b], out_hbm.at[pl.ds(base + off, CHUNK)], wsems[b]
        )

    for b in range(NBUF - 1):
        gather_desc(b, b).start()

    def process(g, b):
        pb = (b + NBUF - 1) % NBUF
        gp = g + NBUF - 1

        @pl.when(gp < NCHUNK)
        def _prefetch():
            @pl.when(gp >= NBUF)
            def _reclaim():
                write_desc(gp - NBUF, pb).wait()
            gather_desc(gp, pb).start()

        gather_desc(g, b).wait()

        off = g * CHUNK
        flags = []
        for v in range(CHUNK // LANES):
            ids = idx_v[pl.ds(off + v * LANES, LANES)]
            mk = mask_v[pl.ds(off + v * LANES, LANES)]
            flags.append(jnp.any((ids == 0) | (mk != 1.0)))
        need = functools.reduce(jnp.logical_or, flags)

        @pl.when(need)
        def _fixup():
            def fix_row(r, cr):
                pos = jnp.full((LANES,), off + r, dtype=jnp.int32)
                idb = plsc.load_gather(idx_v, [pos])
                mkb = plsc.load_gather(mask_v, [pos])
                scale = jnp.where(idb == 0, jnp.zeros((LANES,), jnp.float32), mkb)
                for j in range(D // LANES):
                    bufs[b][r, pl.ds(j * LANES, LANES)] = (
                        bufs[b][r, pl.ds(j * LANES, LANES)] * scale
                    )
                return cr
            lax.fori_loop(0, CHUNK, fix_row, 0)

        write_desc(g, b).start()

    def outer(g2, carry):
        for b in range(NBUF):
            process(g2 * NBUF + b, b)
        return carry

    lax.fori_loop(0, NCHUNK // NBUF, outer, 0)

    for b in range(NBUF):
        write_desc(NCHUNK - NBUF + b, b).wait()


@jax.jit
def _emb_call(ids_flat, mask_flat, table):
    mesh = plsc.VectorSubcoreMesh(core_axis_name="c", subcore_axis_name="s")
    kern = pl.kernel(
        _emb_body,
        out_type=jax.ShapeDtypeStruct((N, D), jnp.float32),
        mesh=mesh,
        compiler_params=pltpu.CompilerParams(needs_layout_passes=False),
        scratch_types=(
            [
                pltpu.VMEM((PER_W,), jnp.int32),
                pltpu.VMEM((PER_W,), jnp.float32),
            ]
            + [pltpu.VMEM((CHUNK, D), jnp.float32) for _ in range(NBUF)]
            + [pltpu.SemaphoreType.DMA for _ in range(2 * NBUF)]
        ),
    )
    return kern(ids_flat, mask_flat, table)


def kernel(input_ids, attention_mask, table):
    B, L = input_ids.shape
    out = _emb_call(
        input_ids.reshape(-1).astype(jnp.int32),
        attention_mask.reshape(-1).astype(jnp.float32),
        table,
    )
    return out.reshape(B, L, D)


# speedup vs baseline: 8.0719x; 1.0105x over previous
import functools

import jax
import jax.numpy as jnp
from jax import lax
from jax.experimental import pallas as pl
from jax.experimental.pallas import tpu as pltpu
from jax.experimental.pallas import tpu_sc as plsc

VOCAB = 100000
D = 128
N = 1024 * 200
NC, NS, LANES = 2, 16, 16
NW = NC * NS
PER_W = N // NW
CHUNK = 128
NCHUNK = PER_W // CHUNK
NBUF = 5


def _emb_body(ids_hbm, mask_hbm, table_hbm, out_hbm, idx_v, mask_v, *rest):
    bufs = rest[:NBUF]
    gsems = rest[NBUF:2 * NBUF]
    wsems = rest[2 * NBUF:3 * NBUF]
    msem = rest[3 * NBUF]

    c = lax.axis_index("c")
    s = lax.axis_index("s")
    wid = s * NC + c
    base = wid * PER_W

    pltpu.sync_copy(ids_hbm.at[pl.ds(base, PER_W)], idx_v)
    mask_cp = pltpu.async_copy(mask_hbm.at[pl.ds(base, PER_W)], mask_v, msem)

    def gather_desc(g, b):
        off = g * CHUNK
        return pltpu.make_async_copy(
            table_hbm.at[idx_v.at[pl.ds(off, CHUNK)]], bufs[b], gsems[b]
        )

    def write_desc(g, b):
        off = g * CHUNK
        return pltpu.make_async_copy(
            bufs[b], out_hbm.at[pl.ds(base + off, CHUNK)], wsems[b]
        )

    for b in range(NBUF - 1):
        gather_desc(b, b).start()
    mask_cp.wait()

    def process(g, b):
        pb = (b + NBUF - 1) % NBUF
        gp = g + NBUF - 1

        @pl.when(gp < NCHUNK)
        def _prefetch():
            @pl.when(gp >= NBUF)
            def _reclaim():
                write_desc(gp - NBUF, pb).wait()
            gather_desc(gp, pb).start()

        gather_desc(g, b).wait()

        off = g * CHUNK
        flags = []
        for v in range(CHUNK // LANES):
            ids = idx_v[pl.ds(off + v * LANES, LANES)]
            mk = mask_v[pl.ds(off + v * LANES, LANES)]
            flags.append(jnp.any((ids == 0) | (mk != 1.0)))
        need = functools.reduce(jnp.logical_or, flags)

        @pl.when(need)
        def _fixup():
            def fix_row(r, cr):
                pos = jnp.full((LANES,), off + r, dtype=jnp.int32)
                idb = plsc.load_gather(idx_v, [pos])
                mkb = plsc.load_gather(mask_v, [pos])
                scale = jnp.where(idb == 0, jnp.zeros((LANES,), jnp.float32), mkb)
                for j in range(D // LANES):
                    bufs[b][r, pl.ds(j * LANES, LANES)] = (
                        bufs[b][r, pl.ds(j * LANES, LANES)] * scale
                    )
                return cr
            lax.fori_loop(0, CHUNK, fix_row, 0)

        write_desc(g, b).start()

    def outer(g2, carry):
        for b in range(NBUF):
            process(g2 * NBUF + b, b)
        return carry

    lax.fori_loop(0, NCHUNK // NBUF, outer, 0)

    for b in range(NBUF):
        write_desc(NCHUNK - NBUF + b, b).wait()


@jax.jit
def _emb_call(ids_flat, mask_flat, table):
    mesh = plsc.VectorSubcoreMesh(core_axis_name="c", subcore_axis_name="s")
    kern = pl.kernel(
        _emb_body,
        out_type=jax.ShapeDtypeStruct((N, D), jnp.float32),
        mesh=mesh,
        compiler_params=pltpu.CompilerParams(
            needs_layout_passes=False,
            disable_bounds_checks=True,
            disable_semaphore_checks=True,
        ),
        scratch_types=(
            [
                pltpu.VMEM((PER_W,), jnp.int32),
                pltpu.VMEM((PER_W,), jnp.float32),
            ]
            + [pltpu.VMEM((CHUNK, D), jnp.float32) for _ in range(NBUF)]
            + [pltpu.SemaphoreType.DMA for _ in range(2 * NBUF + 1)]
        ),
    )
    return kern(ids_flat, mask_flat, table)


def kernel(input_ids, attention_mask, table):
    B, L = input_ids.shape
    out = _emb_call(
        input_ids.reshape(-1).astype(jnp.int32),
        attention_mask.reshape(-1).astype(jnp.float32),
        table,
    )
    return out.reshape(B, L, D)


# speedup vs baseline: 8.0960x; 1.0030x over previous
import functools

import jax
import jax.numpy as jnp
from jax import lax
from jax.experimental import pallas as pl
from jax.experimental.pallas import tpu as pltpu
from jax.experimental.pallas import tpu_sc as plsc

VOCAB = 100000
D = 128
N = 1024 * 200
NC, NS, LANES = 2, 16, 16
NW = NC * NS
PER_W = N // NW
CHUNK = 64
NCHUNK = PER_W // CHUNK
NBUF = 10


def _emb_body(ids_hbm, mask_hbm, table_hbm, out_hbm, idx_v, mask_v, *rest):
    bufs = rest[:NBUF]
    gsems = rest[NBUF:2 * NBUF]
    wsems = rest[2 * NBUF:3 * NBUF]
    msem = rest[3 * NBUF]

    c = lax.axis_index("c")
    s = lax.axis_index("s")
    wid = s * NC + c
    base = wid * PER_W

    pltpu.sync_copy(ids_hbm.at[pl.ds(base, PER_W)], idx_v)
    mask_cp = pltpu.async_copy(mask_hbm.at[pl.ds(base, PER_W)], mask_v, msem)

    def gather_desc(g, b):
        off = g * CHUNK
        return pltpu.make_async_copy(
            table_hbm.at[idx_v.at[pl.ds(off, CHUNK)]], bufs[b], gsems[b]
        )

    def write_desc(g, b):
        off = g * CHUNK
        return pltpu.make_async_copy(
            bufs[b], out_hbm.at[pl.ds(base + off, CHUNK)], wsems[b]
        )

    for b in range(NBUF - 1):
        gather_desc(b, b).start()
    mask_cp.wait()

    def process(g, b):
        pb = (b + NBUF - 1) % NBUF
        gp = g + NBUF - 1

        @pl.when(gp < NCHUNK)
        def _prefetch():
            @pl.when(gp >= NBUF)
            def _reclaim():
                write_desc(gp - NBUF, pb).wait()
            gather_desc(gp, pb).start()

        off = g * CHUNK
        flags = []
        for v in range(CHUNK // LANES):
            ids = idx_v[pl.ds(off + v * LANES, LANES)]
            mk = mask_v[pl.ds(off + v * LANES, LANES)]
            flags.append(jnp.any((ids == 0) | (mk != 1.0)))
        need = functools.reduce(jnp.logical_or, flags)

        gather_desc(g, b).wait()

        @pl.when(need)
        def _fixup():
            def fix_row(r, cr):
                pos = jnp.full((LANES,), off + r, dtype=jnp.int32)
                idb = plsc.load_gather(idx_v, [pos])
                mkb = plsc.load_gather(mask_v, [pos])
                scale = jnp.where(idb == 0, jnp.zeros((LANES,), jnp.float32), mkb)
                for j in range(D // LANES):
                    bufs[b][r, pl.ds(j * LANES, LANES)] = (
                        bufs[b][r, pl.ds(j * LANES, LANES)] * scale
                    )
                return cr
            lax.fori_loop(0, CHUNK, fix_row, 0)

        write_desc(g, b).start()

    def outer(g2, carry):
        for b in range(NBUF):
            process(g2 * NBUF + b, b)
        return carry

    lax.fori_loop(0, NCHUNK // NBUF, outer, 0)

    for b in range(NBUF):
        write_desc(NCHUNK - NBUF + b, b).wait()


@jax.jit
def _emb_call(ids_flat, mask_flat, table):
    mesh = plsc.VectorSubcoreMesh(core_axis_name="c", subcore_axis_name="s")
    kern = pl.kernel(
        _emb_body,
        out_type=jax.ShapeDtypeStruct((N, D), jnp.float32),
        mesh=mesh,
        compiler_params=pltpu.CompilerParams(
            needs_layout_passes=False,
            disable_bounds_checks=True,
            disable_semaphore_checks=True,
        ),
        scratch_types=(
            [
                pltpu.VMEM((PER_W,), jnp.int32),
                pltpu.VMEM((PER_W,), jnp.float32),
            ]
            + [pltpu.VMEM((CHUNK, D), jnp.float32) for _ in range(NBUF)]
            + [pltpu.SemaphoreType.DMA for _ in range(2 * NBUF + 1)]
        ),
    )
    return kern(ids_flat, mask_flat, table)


def kernel(input_ids, attention_mask, table):
    B, L = input_ids.shape
    out = _emb_call(
        input_ids.reshape(-1).astype(jnp.int32),
        attention_mask.reshape(-1).astype(jnp.float32),
        table,
    )
    return out.reshape(B, L, D)
